# baseline (speedup 1.0000x reference)
import functools

import jax
import jax.numpy as jnp
from jax import lax
from jax.experimental import pallas as pl
from jax.experimental.pallas import tpu as pltpu
from jax.experimental.pallas import tpu_sc as plsc

SIZE = 1024
DIM = 64
PDIM = 128
N = 32 * 576
BLK = 512
NBLK = N // BLK

NW = 32
RPW = N // NW
CHUNK = 96
NCHUNK = RPW // CHUNK


def _tc_body(z_ref, cb_ref, zn_ref, cn_ref, idx_ref, perp_ref, counts_ref):
    i = pl.program_id(0)

    @pl.when(i == 0)
    def _init():
        counts_ref[...] = jnp.zeros_like(counts_ref)

    z16 = z_ref[...].astype(jnp.bfloat16)
    cb16 = cb_ref[...].astype(jnp.bfloat16)
    scores = lax.dot_general(z16, cb16, (((1,), (1,)), ((), ())),
                             preferred_element_type=jnp.float32)
    cost = (zn_ref[...] + cn_ref[...]) - 2.0 * scores
    idx = jnp.argmin(cost, axis=1).astype(jnp.int32)
    idx_ref[...] = idx

    onehot = (lax.broadcasted_iota(jnp.int32, (BLK, SIZE), 1)
              == idx[:, None]).astype(jnp.float32)
    counts_ref[...] += jnp.sum(onehot, axis=0, keepdims=True)

    @pl.when(i == NBLK - 1)
    def _fin():
        e = counts_ref[...] * (1.0 / N)
        perp_ref[0, 0] = jnp.exp(-jnp.sum(e * jnp.log(e + 1e-10)))


def _tc_argmin(z_flat, codebook, znorm, cnorm, interpret=False):
    return pl.pallas_call(
        _tc_body,
        grid=(NBLK,),
        in_specs=[
            pl.BlockSpec((BLK, DIM), lambda i: (i, 0)),
            pl.BlockSpec((SIZE, DIM), lambda i: (0, 0)),
            pl.BlockSpec((BLK, 1), lambda i: (i, 0)),
            pl.BlockSpec((1, SIZE), lambda i: (0, 0)),
        ],
        out_specs=[
            pl.BlockSpec((BLK,), lambda i: (i,)),
            pl.BlockSpec(memory_space=pltpu.SMEM),
        ],
        out_shape=[
            jax.ShapeDtypeStruct((N,), jnp.int32),
            jax.ShapeDtypeStruct((1, 1), jnp.float32),
        ],
        scratch_shapes=[pltpu.VMEM((1, SIZE), jnp.float32)],
        compiler_params=pltpu.CompilerParams(
            dimension_semantics=("arbitrary",)),
        interpret=interpret,
    )(z_flat, codebook, znorm, cnorm)


def _sc_gather_build():
    mesh = plsc.VectorSubcoreMesh(core_axis_name="c", subcore_axis_name="s")

    @functools.partial(
        pl.kernel,
        mesh=mesh,
        out_type=jax.ShapeDtypeStruct((N, PDIM), jnp.float32),
        scratch_types=[
            pltpu.VMEM((RPW,), jnp.int32),
            pltpu.VMEM((RPW, PDIM), jnp.float32),
            pltpu.SemaphoreType.DMA,
        ],
    )
    def _sc_gather(cb_hbm, idx_hbm, out_hbm, idx_v, rows_v, sem):
        wid = lax.axis_index("s") * 2 + lax.axis_index("c")
        base = wid * RPW
        pltpu.sync_copy(idx_hbm.at[pl.ds(base, RPW)], idx_v)
        copies = []
        for j in range(NCHUNK):
            copies.append(pltpu.async_copy(
                cb_hbm.at[idx_v.at[pl.ds(j * CHUNK, CHUNK)]],
                rows_v.at[pl.ds(j * CHUNK, CHUNK)],
                sem))
        for c in copies:
            c.wait()
        pltpu.sync_copy(rows_v, out_hbm.at[pl.ds(base, RPW)])

    return _sc_gather


def kernel(z_from_encoder, codebook, codebook_weight, flg_train):
    z = z_from_encoder
    z_flat = z.reshape(-1, DIM)
    znorm = jnp.sum(z_flat ** 2, axis=1, keepdims=True)
    cnorm = jnp.sum(codebook ** 2, axis=1)[None, :]
    idx, perp = _tc_argmin(z_flat, codebook, znorm, cnorm)
    cb_pad = jnp.pad(codebook, ((0, 0), (0, PDIM - DIM)))
    z_q = _sc_gather_build()(cb_pad, idx)
    z_q = z_q[:, :DIM].reshape(z.shape)
    return (z_q, 0.0, perp[0, 0])


# speedup vs baseline: 1.0989x; 1.0989x over previous
import functools

import jax
import jax.numpy as jnp
from jax import lax
from jax.experimental import pallas as pl
from jax.experimental.pallas import tpu as pltpu
from jax.experimental.pallas import tpu_sc as plsc

SIZE = 1024
DIM = 64
N = 32 * 576
BLK = 512
NBLK = N // BLK

NW = 32
RPW = N // NW
CHUNK = 96
NCHUNK = RPW // CHUNK


def _tc_body(z_ref, cb_ref, cn_ref, idx_ref, perp_ref, counts_ref):
    i = pl.program_id(0)

    @pl.when(i == 0)
    def _init():
        counts_ref[...] = jnp.zeros_like(counts_ref)

    z = z_ref[...]
    z16 = z.astype(jnp.bfloat16)
    cb16 = cb_ref[...].astype(jnp.bfloat16)
    scores = lax.dot_general(z16, cb16, (((1,), (1,)), ((), ())),
                             preferred_element_type=jnp.float32)
    zn = jnp.sum(z * z, axis=1, keepdims=True)
    cost = (zn + cn_ref[...]) - 2.0 * scores
    idx = jnp.argmin(cost, axis=1).astype(jnp.int32)
    idx_ref[...] = idx

    onehot = (lax.broadcasted_iota(jnp.int32, (BLK, SIZE), 1)
              == idx[:, None]).astype(jnp.float32)
    counts_ref[...] += jnp.sum(onehot, axis=0, keepdims=True)

    @pl.when(i == NBLK - 1)
    def _fin():
        e = counts_ref[...] * (1.0 / N)
        perp_ref[0, 0] = jnp.exp(-jnp.sum(e * jnp.log(e + 1e-10)))


def _tc_argmin(z_flat, codebook, cnorm, interpret=False):
    return pl.pallas_call(
        _tc_body,
        grid=(NBLK,),
        in_specs=[
            pl.BlockSpec((BLK, DIM), lambda i: (i, 0)),
            pl.BlockSpec((SIZE, DIM), lambda i: (0, 0)),
            pl.BlockSpec((1, SIZE), lambda i: (0, 0)),
        ],
        out_specs=[
            pl.BlockSpec((BLK,), lambda i: (i,)),
            pl.BlockSpec(memory_space=pltpu.SMEM),
        ],
        out_shape=[
            jax.ShapeDtypeStruct((N,), jnp.int32),
            jax.ShapeDtypeStruct((1, 1), jnp.float32),
        ],
        scratch_shapes=[pltpu.VMEM((1, SIZE), jnp.float32)],
        compiler_params=pltpu.CompilerParams(
            dimension_semantics=("arbitrary",)),
        interpret=interpret,
    )(z_flat, codebook, cnorm)


def _sc_gather_build():
    mesh = plsc.VectorSubcoreMesh(core_axis_name="c", subcore_axis_name="s")

    @functools.partial(
        pl.kernel,
        mesh=mesh,
        out_type=jax.ShapeDtypeStruct((N, DIM), jnp.float32),
        scratch_types=[
            pltpu.VMEM((RPW,), jnp.int32),
            pltpu.VMEM((RPW, DIM), jnp.float32),
            pltpu.SemaphoreType.DMA,
        ],
        compiler_params=pltpu.CompilerParams(use_tc_tiling_on_sc=False),
    )
    def _sc_gather(cb_hbm, idx_hbm, out_hbm, idx_v, rows_v, sem):
        wid = lax.axis_index("s") * 2 + lax.axis_index("c")
        base = wid * RPW
        pltpu.sync_copy(idx_hbm.at[pl.ds(base, RPW)], idx_v)
        copies = []
        for j in range(NCHUNK):
            copies.append(pltpu.async_copy(
                cb_hbm.at[idx_v.at[pl.ds(j * CHUNK, CHUNK)]],
                rows_v.at[pl.ds(j * CHUNK, CHUNK)],
                sem))
        for c in copies:
            c.wait()
        pltpu.sync_copy(rows_v, out_hbm.at[pl.ds(base, RPW)])

    return _sc_gather


def kernel(z_from_encoder, codebook, codebook_weight, flg_train):
    z = z_from_encoder
    z_flat = z.reshape(-1, DIM)
    cnorm = jnp.sum(codebook ** 2, axis=1)[None, :]
    idx, perp = _tc_argmin(z_flat, codebook, cnorm)
    z_q = _sc_gather_build()(codebook, idx)
    z_q = z_q.reshape(z.shape)
    return (z_q, 0.0, perp[0, 0])
